# Initial kernel scaffold; baseline (speedup 1.0000x reference)
import jax
import jax.numpy as jnp
from jax.experimental import pallas as pl


def kernel(x, FM, prelu_a):
    raise NotImplementedError("write your pallas kernel here")


# speedup vs baseline: 827.1125x; 827.1125x over previous
import functools

import jax
import jax.numpy as jnp
from jax import lax
from jax.experimental import pallas as pl
from jax.experimental.pallas import tpu as pltpu
from jax.experimental.pallas import tpu_sc as plsc

N_NODES = 50000
S_NEIGH = 8
D_FEAT = 128
LANES = 16
NW = 32
NB = 40
NBLK = N_NODES // NB
DC = D_FEAT // LANES

_PAIRS = (
    (0, 1), (2, 3), (4, 5), (6, 7),
    (0, 2), (1, 3), (4, 6), (5, 7),
    (1, 2), (5, 6),
    (0, 4), (1, 5), (2, 6), (3, 7),
    (2, 4), (3, 5),
    (1, 2), (3, 4), (5, 6),
)


def _choquet_node(xb, ob, fmt, g_last, n):
    for dc in range(DC):
        ks = [xb[n, j, pl.ds(dc * LANES, LANES)] for j in range(S_NEIGH)]
        ps = [jnp.full((LANES,), 1 << j, jnp.int32) for j in range(S_NEIGH)]
        for (i, j) in _PAIRS:
            m = ks[j] > ks[i]
            ki = jnp.where(m, ks[j], ks[i])
            kj = jnp.where(m, ks[i], ks[j])
            pi = jnp.where(m, ps[j], ps[i])
            pj = jnp.where(m, ps[i], ps[j])
            ks[i], ks[j], ps[i], ps[j] = ki, kj, pi, pj
        c = ps[0]
        acc = plsc.load_gather(fmt, [c]) * (ks[0] - ks[1])
        for i in range(1, S_NEIGH):
            c = c + ps[i]
            if i < S_NEIGH - 1:
                g = plsc.load_gather(fmt, [c])
                acc = acc + g * (ks[i] - ks[i + 1])
            else:
                acc = acc + g_last * ks[i]
        ob[n, pl.ds(dc * LANES, LANES)] = acc


_MESH = plsc.VectorSubcoreMesh(core_axis_name="c", subcore_axis_name="s")


@functools.partial(
    pl.kernel,
    mesh=_MESH,
    out_type=jax.ShapeDtypeStruct((N_NODES, D_FEAT), jnp.float32),
    scratch_types=[
        pltpu.VMEM((NB, S_NEIGH, D_FEAT), jnp.float32),
        pltpu.VMEM((NB, D_FEAT), jnp.float32),
        pltpu.VMEM((256,), jnp.float32),
        pltpu.VMEM((LANES,), jnp.float32),
    ],
    compiler_params=pltpu.CompilerParams(needs_layout_passes=False),
)
def _choquet_sc(x_hbm, fm_hbm, a_hbm, out_hbm, xb, ob, fmt, av):
    wid = lax.axis_index("s") * 2 + lax.axis_index("c")

    pltpu.sync_copy(fm_hbm, fmt)
    pltpu.sync_copy(a_hbm, av)
    a = av[...]
    for cch in range(256 // LANES):
        v = fmt[pl.ds(cch * LANES, LANES)]
        fmt[pl.ds(cch * LANES, LANES)] = jnp.where(v >= 0.0, v, a * v)
    g_last = plsc.load_gather(fmt, [jnp.full((LANES,), 255, jnp.int32)])

    nloops = (NBLK + NW - 1) // NW

    def blk_body(t, carry):
        blk = t * NW + wid

        @pl.when(blk < NBLK)
        def _():
            base = blk * NB
            pltpu.sync_copy(x_hbm.at[pl.ds(base, NB)], xb)

            def node_body(n, c2):
                _choquet_node(xb, ob, fmt, g_last, n)
                return c2

            lax.fori_loop(0, NB, node_body, 0)
            pltpu.sync_copy(ob, out_hbm.at[pl.ds(base, NB)])

        return carry

    lax.fori_loop(0, nloops, blk_body, 0)


def kernel(x, FM, prelu_a):
    fm_pad = jnp.concatenate([jnp.zeros((1,), jnp.float32), FM[:, 0]])
    a_vec = jnp.full((LANES,), prelu_a, dtype=jnp.float32)
    return _choquet_sc(x, fm_pad, a_vec)


# speedup vs baseline: 960.7679x; 1.1616x over previous
import functools

import jax
import jax.numpy as jnp
from jax import lax
from jax.experimental import pallas as pl
from jax.experimental.pallas import tpu as pltpu
from jax.experimental.pallas import tpu_sc as plsc

N_NODES = 50000
S_NEIGH = 8
D_FEAT = 128
LANES = 16
NW = 32
NB = 40
NBLK = N_NODES // NB
DC = D_FEAT // LANES
NLOOPS = (NBLK + NW - 1) // NW

_PAIRS = (
    (0, 1), (2, 3), (4, 5), (6, 7),
    (0, 2), (1, 3), (4, 6), (5, 7),
    (1, 2), (5, 6),
    (0, 4), (1, 5), (2, 6), (3, 7),
    (2, 4), (3, 5),
    (1, 2), (3, 4), (5, 6),
)


def _choquet_node(xb, ob, fmt, g_last, n):
    for dc in range(DC):
        ks = [xb[n, j, pl.ds(dc * LANES, LANES)] for j in range(S_NEIGH)]
        ps = [jnp.full((LANES,), 1 << j, jnp.int32) for j in range(S_NEIGH)]
        for (i, j) in _PAIRS:
            m = ks[j] > ks[i]
            ki = jnp.where(m, ks[j], ks[i])
            kj = jnp.where(m, ks[i], ks[j])
            pi = jnp.where(m, ps[j], ps[i])
            pj = jnp.where(m, ps[i], ps[j])
            ks[i], ks[j], ps[i], ps[j] = ki, kj, pi, pj
        c = ps[0]
        acc = plsc.load_gather(fmt, [c]) * (ks[0] - ks[1])
        for i in range(1, S_NEIGH):
            c = c + ps[i]
            if i < S_NEIGH - 1:
                g = plsc.load_gather(fmt, [c])
                acc = acc + g * (ks[i] - ks[i + 1])
            else:
                acc = acc + g_last * ks[i]
        ob[n, pl.ds(dc * LANES, LANES)] = acc


_MESH = plsc.VectorSubcoreMesh(core_axis_name="c", subcore_axis_name="s")


@functools.partial(
    pl.kernel,
    mesh=_MESH,
    out_type=jax.ShapeDtypeStruct((N_NODES, D_FEAT), jnp.float32),
    scratch_types=[
        pltpu.VMEM((2, NB, S_NEIGH, D_FEAT), jnp.float32),
        pltpu.VMEM((2, NB, D_FEAT), jnp.float32),
        pltpu.VMEM((256,), jnp.float32),
        pltpu.VMEM((LANES,), jnp.float32),
        pltpu.SemaphoreType.DMA,
        pltpu.SemaphoreType.DMA,
        pltpu.SemaphoreType.DMA,
        pltpu.SemaphoreType.DMA,
    ],
    compiler_params=pltpu.CompilerParams(needs_layout_passes=False),
)
def _choquet_sc(x_hbm, fm_hbm, a_hbm, out_hbm, xb, ob, fmt, av,
                sin0, sin1, sout0, sout1):
    wid = lax.axis_index("s") * 2 + lax.axis_index("c")

    pltpu.sync_copy(fm_hbm, fmt)
    pltpu.sync_copy(a_hbm, av)
    a = av[...]
    for cch in range(256 // LANES):
        v = fmt[pl.ds(cch * LANES, LANES)]
        fmt[pl.ds(cch * LANES, LANES)] = jnp.where(v >= 0.0, v, a * v)
    g_last = plsc.load_gather(fmt, [jnp.full((LANES,), 255, jnp.int32)])

    sin = (sin0, sin1)
    sout = (sout0, sout1)

    def start_in(t, b):
        @pl.when(t * NW + wid < NBLK)
        def _():
            pltpu.async_copy(
                x_hbm.at[pl.ds((t * NW + wid) * NB, NB)], xb.at[b], sin[b])

    def half(t, b):
        blk = t * NW + wid

        @pl.when((t >= 2) & (blk - 2 * NW < NBLK))
        def _():
            pltpu.make_async_copy(
                ob.at[b], out_hbm.at[pl.ds((blk - 2 * NW) * NB, NB)],
                sout[b]).wait()

        @pl.when(blk < NBLK)
        def _():
            pltpu.make_async_copy(
                x_hbm.at[pl.ds(blk * NB, NB)], xb.at[b], sin[b]).wait()

            def node_body(n, c2):
                _choquet_node(xb.at[b], ob.at[b], fmt, g_last, n)
                return c2

            lax.fori_loop(0, NB, node_body, 0)
            pltpu.async_copy(ob.at[b], out_hbm.at[pl.ds(blk * NB, NB)],
                             sout[b])

    start_in(jnp.int32(0), 0)

    def pair_body(i, carry):
        t0 = 2 * i
        start_in(t0 + 1, 1)
        half(t0, 0)
        start_in(t0 + 2, 0)
        half(t0 + 1, 1)
        return carry

    lax.fori_loop(0, NLOOPS // 2, pair_body, 0)

    for t in (NLOOPS - 2, NLOOPS - 1):
        @pl.when(t * NW + wid < NBLK)
        def _():
            pltpu.make_async_copy(
                ob.at[t % 2], out_hbm.at[pl.ds((t * NW + wid) * NB, NB)],
                sout[t % 2]).wait()


def kernel(x, FM, prelu_a):
    fm_pad = jnp.concatenate([jnp.zeros((1,), jnp.float32), FM[:, 0]])
    a_vec = jnp.full((LANES,), prelu_a, dtype=jnp.float32)
    return _choquet_sc(x, fm_pad, a_vec)
